# baseline (speedup 1.0000x reference)
import functools

import jax
import jax.numpy as jnp
from jax import lax
from jax.experimental import pallas as pl
from jax.experimental.pallas import tpu as pltpu
from jax.experimental.pallas import tpu_sc as plsc

_VOCAB = 1000000
_DIM = 32
_BATCH = 4096
_HIST = 50

_NC = 2
_NS = 16
_NW = _NC * _NS
_NBT = _BATCH // 128
_UNITS = _HIST * _NBT
_UPW = _UNITS // _NW

_NJ = _VOCAB // 128
_JPW = _NJ // _NW
_NJR = _NJ - _JPW * _NW
_VPAD = 1000064

_mesh = plsc.VectorSubcoreMesh(core_axis_name="c", subcore_axis_name="s")


@functools.partial(
    pl.kernel,
    out_type=jax.ShapeDtypeStruct((_VPAD * 2 * _DIM,), jnp.float32),
    mesh=_mesh,
    scratch_types=[
        pltpu.VMEM((_DIM, 128), jnp.float32),
        pltpu.VMEM((_DIM, 128), jnp.float32),
        pltpu.VMEM((128 * 2 * _DIM,), jnp.float32),
        pltpu.SemaphoreType.DMA,
    ],
    compiler_params=pltpu.CompilerParams(
        use_tc_tiling_on_sc=True, needs_layout_passes=False),
)
def _format(tr_hbm, ti_hbm, ct_hbm, buf_r, buf_i, ct_buf, sem):
    wid = lax.axis_index("s") * _NC + lax.axis_index("c")
    iota16 = lax.iota(jnp.int32, 16)
    dlanes = [iota16 + 16 * dh for dh in range(_DIM // 16)]
    nj = _JPW + jnp.where(wid < _NJR + 1, 1, 0)

    def block(k, carry):
        j = jnp.where(k < _JPW, wid * _JPW + k, _JPW * _NW + wid)
        partial = j >= _NJ
        v0 = pl.multiple_of(jnp.minimum(j, _NJ) * 128, 128)
        cp_r = pltpu.async_copy(tr_hbm.at[:, pl.ds(v0, 128)], buf_r, sem)
        cp_i = pltpu.async_copy(ti_hbm.at[:, pl.ds(v0, 128)], buf_i, sem)
        cp_r.wait()
        cp_i.wait()

        def tcol(c8, tc):
            c0 = c8 * 8
            for ci in range(8):
                c = c0 + ci
                base = c * (2 * _DIM)
                cvec = jnp.full((16,), c, dtype=jnp.int32)
                for dhalf in range(_DIM // 16):
                    vr = plsc.load_gather(buf_r, [dlanes[dhalf], cvec])
                    vi = plsc.load_gather(buf_i, [dlanes[dhalf], cvec])
                    ct_buf[pl.ds(base + 16 * dhalf, 16)] = vr
                    ct_buf[pl.ds(base + 16 * dhalf + _DIM, 16)] = vi
            return tc

        lax.fori_loop(0, 16, tcol, 0)
        half = 64 * 2 * _DIM
        dst0 = v0 * 2 * _DIM
        pltpu.sync_copy(ct_buf.at[pl.ds(0, half)],
                        ct_hbm.at[pl.ds(dst0, half)])

        @pl.when(jnp.logical_not(partial))
        def _():
            pltpu.sync_copy(ct_buf.at[pl.ds(half, half)],
                            ct_hbm.at[pl.ds(dst0 + half, half)])
        return carry

    lax.fori_loop(0, nj, block, 0)


@functools.partial(
    pl.kernel,
    out_type=(
        jax.ShapeDtypeStruct((_HIST, _DIM // 8, _NBT, 8, 128), jnp.float32),
        jax.ShapeDtypeStruct((_HIST, _DIM // 8, _NBT, 8, 128), jnp.float32),
    ),
    mesh=_mesh,
    scratch_types=[
        pltpu.VMEM((128,), jnp.int32),
        pltpu.VMEM((128, 2 * _DIM), jnp.float32),
        pltpu.VMEM((_DIM, 128), jnp.float32),
        pltpu.VMEM((_DIM, 128), jnp.float32),
        pltpu.SemaphoreType.DMA,
        pltpu.SemaphoreType.DMA,
    ],
    compiler_params=pltpu.CompilerParams(
        use_tc_tiling_on_sc=False, needs_layout_passes=False),
)
def _gather(ids_hbm, ct_hbm, out_r_hbm, out_i_hbm,
            idx_v, rows, out_tr, out_ti, sem_g, sem_o):
    wid = lax.axis_index("s") * _NC + lax.axis_index("c")
    ubase = wid * _UPW
    iota16 = lax.iota(jnp.int32, 16)
    dlanes = [iota16 + 16 * dh for dh in range(_DIM // 16)]

    def unit(k, carry):
        u = ubase + k
        h = u // _NBT
        bt = u % _NBT
        pltpu.sync_copy(ids_hbm.at[h, pl.ds(bt * 128, 128)], idx_v)
        pltpu.async_copy(ct_hbm.at[idx_v], rows, sem_g).wait()

        def trow(c8, tc):
            c0 = c8 * 8
            for ci in range(8):
                c = c0 + ci
                cvec = jnp.full((16,), c, dtype=jnp.int32)
                for dhalf in range(_DIM // 16):
                    vr = rows[c, pl.ds(16 * dhalf, 16)]
                    vi = rows[c, pl.ds(16 * dhalf + _DIM, 16)]
                    plsc.store_scatter(out_tr, [dlanes[dhalf], cvec], vr)
                    plsc.store_scatter(out_ti, [dlanes[dhalf], cvec], vi)
            return tc

        lax.fori_loop(0, 16, trow, 0)
        ocps = []
        for dt in range(_DIM // 8):
            ocps.append(pltpu.async_copy(
                out_tr.at[pl.ds(dt * 8, 8)], out_r_hbm.at[h, dt, bt], sem_o))
            ocps.append(pltpu.async_copy(
                out_ti.at[pl.ds(dt * 8, 8)], out_i_hbm.at[h, dt, bt], sem_o))
        for cp in ocps:
            cp.wait()
        return carry

    lax.fori_loop(0, _UPW, unit, 0)


def kernel(input_ids, emb_real, emb_imag):
    ids_t = input_ids.T.astype(jnp.int32)
    ct_flat = _format(emb_real.T, emb_imag.T)
    ct = ct_flat.reshape(_VPAD, 2 * _DIM)
    o_r, o_i = _gather(ids_t, ct)
    p_r = o_r.transpose(0, 1, 3, 2, 4).reshape(_HIST, _DIM, _BATCH)
    p_i = o_i.transpose(0, 1, 3, 2, 4).reshape(_HIST, _DIM, _BATCH)
    out_t = lax.complex(p_r, p_i)
    return out_t.transpose(2, 0, 1)


# speedup vs baseline: 1.2128x; 1.2128x over previous
import functools

import jax
import jax.numpy as jnp
from jax import lax
from jax.experimental import pallas as pl
from jax.experimental.pallas import tpu as pltpu
from jax.experimental.pallas import tpu_sc as plsc

_VOCAB = 1000000
_DIM = 32
_BATCH = 4096
_HIST = 50

_NC = 2
_NS = 16
_NW = _NC * _NS
_NBT = _BATCH // 128
_UNITS = _HIST * _NBT
_UPW = _UNITS // _NW

_NJ = _VOCAB // 128
_JPW = _NJ // _NW
_NJR = _NJ - _JPW * _NW
_WPW = _JPW // 2
_VPAD = 1000064
_CTW = 256 * 2 * _DIM

_mesh = plsc.VectorSubcoreMesh(core_axis_name="c", subcore_axis_name="s")


@functools.partial(
    pl.kernel,
    out_type=jax.ShapeDtypeStruct((_VPAD * 2 * _DIM,), jnp.float32),
    mesh=_mesh,
    scratch_types=[
        pltpu.VMEM((_DIM, 256), jnp.float32),
        pltpu.VMEM((_DIM, 256), jnp.float32),
        pltpu.VMEM((_DIM, 256), jnp.float32),
        pltpu.VMEM((_DIM, 256), jnp.float32),
        pltpu.VMEM((_CTW,), jnp.float32),
        pltpu.VMEM((_CTW,), jnp.float32),
        pltpu.SemaphoreType.DMA,
        pltpu.SemaphoreType.DMA,
    ],
    compiler_params=pltpu.CompilerParams(
        use_tc_tiling_on_sc=True, needs_layout_passes=False),
)
def _format(tr_hbm, ti_hbm, ct_hbm,
            br0, bi0, br1, bi1, cb0, cb1, sem_in, sem_out):
    wid = lax.axis_index("s") * _NC + lax.axis_index("c")
    iota16 = lax.iota(jnp.int32, 16)
    dlanes = [iota16 + 16 * dh for dh in range(_DIM // 16)]
    br = (br0, br1)
    bi = (bi0, bi1)
    cb = (cb0, cb1)

    def win_v0(t):
        return pl.multiple_of((wid * _JPW + t * 2) * 128, 128)

    def fire_in(t, slot):
        v0 = win_v0(t)
        pltpu.async_copy(tr_hbm.at[:, pl.ds(v0, 256)], br[slot], sem_in)
        pltpu.async_copy(ti_hbm.at[:, pl.ds(v0, 256)], bi[slot], sem_in)

    def shuffle(bufr, bufi, ctb, ncols):
        def tcol(c8, tc):
            c0 = c8 * 8
            for ci in range(8):
                c = c0 + ci
                base = c * (2 * _DIM)
                cvec = jnp.full((16,), c, dtype=jnp.int32)
                for dh in range(_DIM // 16):
                    vr = plsc.load_gather(bufr, [dlanes[dh], cvec])
                    vi = plsc.load_gather(bufi, [dlanes[dh], cvec])
                    ctb[pl.ds(base + 16 * dh, 16)] = vr
                    ctb[pl.ds(base + 16 * dh + _DIM, 16)] = vi
            return tc

        lax.fori_loop(0, ncols // 8, tcol, 0)

    fire_in(0, 0)

    def phase(t, slot):
        @pl.when(t + 1 < _WPW)
        def _():
            fire_in(t + 1, 1 - slot)

        pltpu.make_async_copy(
            tr_hbm.at[:, pl.ds(0, 256)], br[slot], sem_in).wait()
        pltpu.make_async_copy(
            tr_hbm.at[:, pl.ds(0, 256)], bi[slot], sem_in).wait()

        @pl.when(t >= 2)
        def _():
            pltpu.make_async_copy(
                cb[slot], ct_hbm.at[pl.ds(0, _CTW)], sem_out).wait()

        shuffle(br[slot], bi[slot], cb[slot], 256)
        pltpu.async_copy(
            cb[slot], ct_hbm.at[pl.ds(win_v0(t) * 2 * _DIM, _CTW)], sem_out)

    def pair(p, carry):
        phase(2 * p, 0)
        phase(2 * p + 1, 1)
        return carry

    lax.fori_loop(0, _WPW // 2, pair, 0)
    pltpu.make_async_copy(cb0, ct_hbm.at[pl.ds(0, _CTW)], sem_out).wait()
    pltpu.make_async_copy(cb1, ct_hbm.at[pl.ds(0, _CTW)], sem_out).wait()

    @pl.when(wid < _NJR + 1)
    def _():
        j = _JPW * _NW + wid
        partial = j >= _NJ
        v0 = pl.multiple_of(jnp.minimum(j, _NJ) * 128, 128)
        pltpu.sync_copy(tr_hbm.at[:, pl.ds(v0, 128)],
                        br0.at[:, pl.ds(0, 128)])
        pltpu.sync_copy(ti_hbm.at[:, pl.ds(v0, 128)],
                        bi0.at[:, pl.ds(0, 128)])
        shuffle(br0, bi0, cb0, 128)
        half = 64 * 2 * _DIM

        @pl.when(partial)
        def _():
            pltpu.sync_copy(cb0.at[pl.ds(0, half)],
                            ct_hbm.at[pl.ds(v0 * 2 * _DIM, half)])

        @pl.when(jnp.logical_not(partial))
        def _():
            pltpu.sync_copy(cb0.at[pl.ds(0, 2 * half)],
                            ct_hbm.at[pl.ds(v0 * 2 * _DIM, 2 * half)])


@functools.partial(
    pl.kernel,
    out_type=(
        jax.ShapeDtypeStruct((_HIST, _DIM // 8, _NBT, 8, 128), jnp.float32),
        jax.ShapeDtypeStruct((_HIST, _DIM // 8, _NBT, 8, 128), jnp.float32),
    ),
    mesh=_mesh,
    scratch_types=[
        pltpu.VMEM((128,), jnp.int32),
        pltpu.VMEM((128,), jnp.int32),
        pltpu.VMEM((128, 2 * _DIM), jnp.float32),
        pltpu.VMEM((128, 2 * _DIM), jnp.float32),
        pltpu.VMEM((_DIM, 128), jnp.float32),
        pltpu.VMEM((_DIM, 128), jnp.float32),
        pltpu.VMEM((_DIM, 128), jnp.float32),
        pltpu.VMEM((_DIM, 128), jnp.float32),
        pltpu.SemaphoreType.DMA,
        pltpu.SemaphoreType.DMA,
    ],
    compiler_params=pltpu.CompilerParams(
        use_tc_tiling_on_sc=False, needs_layout_passes=False),
)
def _gather(ids_hbm, ct_hbm, out_r_hbm, out_i_hbm,
            idx0, idx1, rows0, rows1, otr0, oti0, otr1, oti1,
            sem_g, sem_o):
    wid = lax.axis_index("s") * _NC + lax.axis_index("c")
    ubase = wid * _UPW
    iota16 = lax.iota(jnp.int32, 16)
    dlanes = [iota16 + 16 * dh for dh in range(_DIM // 16)]
    idx = (idx0, idx1)
    rows = (rows0, rows1)
    otr = (otr0, otr1)
    oti = (oti0, oti1)

    def hb(t):
        u = ubase + t
        return u // _NBT, u % _NBT

    def fire_unit(t, slot):
        h, bt = hb(t)
        pltpu.sync_copy(ids_hbm.at[h, pl.ds(bt * 128, 128)], idx[slot])
        pltpu.async_copy(ct_hbm.at[idx[slot]], rows[slot], sem_g)

    fire_unit(0, 0)

    def phase(t, slot):
        @pl.when(t + 1 < _UPW)
        def _():
            fire_unit(t + 1, 1 - slot)

        pltpu.make_async_copy(ct_hbm.at[idx[slot]], rows[slot], sem_g).wait()

        @pl.when(t >= 2)
        def _():
            for dt in range(_DIM // 8):
                pltpu.make_async_copy(
                    otr[slot].at[pl.ds(dt * 8, 8)],
                    out_r_hbm.at[0, 0, 0], sem_o).wait()
                pltpu.make_async_copy(
                    oti[slot].at[pl.ds(dt * 8, 8)],
                    out_i_hbm.at[0, 0, 0], sem_o).wait()

        rr = rows[slot]
        tr = otr[slot]
        ti = oti[slot]

        def trow(c8, tc):
            c0 = c8 * 8
            for ci in range(8):
                c = c0 + ci
                cvec = jnp.full((16,), c, dtype=jnp.int32)
                for dh in range(_DIM // 16):
                    vr = rr[c, pl.ds(16 * dh, 16)]
                    vi = rr[c, pl.ds(16 * dh + _DIM, 16)]
                    plsc.store_scatter(tr, [dlanes[dh], cvec], vr)
                    plsc.store_scatter(ti, [dlanes[dh], cvec], vi)
            return tc

        lax.fori_loop(0, 16, trow, 0)
        h, bt = hb(t)
        for dt in range(_DIM // 8):
            pltpu.async_copy(tr.at[pl.ds(dt * 8, 8)],
                             out_r_hbm.at[h, dt, bt], sem_o)
            pltpu.async_copy(ti.at[pl.ds(dt * 8, 8)],
                             out_i_hbm.at[h, dt, bt], sem_o)

    def pair(p, carry):
        phase(2 * p, 0)
        phase(2 * p + 1, 1)
        return carry

    lax.fori_loop(0, _UPW // 2, pair, 0)
    for slot in (0, 1):
        for dt in range(_DIM // 8):
            pltpu.make_async_copy(
                otr[slot].at[pl.ds(dt * 8, 8)],
                out_r_hbm.at[0, 0, 0], sem_o).wait()
            pltpu.make_async_copy(
                oti[slot].at[pl.ds(dt * 8, 8)],
                out_i_hbm.at[0, 0, 0], sem_o).wait()


def kernel(input_ids, emb_real, emb_imag):
    ids_t = input_ids.T.astype(jnp.int32)
    ct_flat = _format(emb_real.T, emb_imag.T)
    ct = ct_flat.reshape(_VPAD, 2 * _DIM)
    o_r, o_i = _gather(ids_t, ct)
    p_r = o_r.transpose(0, 1, 3, 2, 4).reshape(_HIST, _DIM, _BATCH)
    p_i = o_i.transpose(0, 1, 3, 2, 4).reshape(_HIST, _DIM, _BATCH)
    out_t = lax.complex(p_r, p_i)
    return out_t.transpose(2, 0, 1)


# speedup vs baseline: 2.0393x; 1.6814x over previous
import functools

import jax
import jax.numpy as jnp
from jax import lax
from jax.experimental import pallas as pl
from jax.experimental.pallas import tpu as pltpu
from jax.experimental.pallas import tpu_sc as plsc

_VOCAB = 1000000
_DIM = 32
_BATCH = 4096
_HIST = 50

_NC = 2
_NS = 16
_NW = _NC * _NS
_NBT = _BATCH // 128
_UNITS = _HIST * _NBT
_UPW = _UNITS // _NW

_NJ = _VOCAB // 128
_JPW = _NJ // _NW
_NJR = _NJ - _JPW * _NW
_WPW = _JPW // 2
_VPAD = 1000064

_mesh = plsc.VectorSubcoreMesh(core_axis_name="c", subcore_axis_name="s")


@functools.partial(
    pl.kernel,
    out_type=jax.ShapeDtypeStruct((_VPAD, 2 * _DIM), jnp.float32),
    mesh=_mesh,
    scratch_types=[
        pltpu.VMEM((_DIM, 256), jnp.float32),
        pltpu.VMEM((_DIM, 256), jnp.float32),
        pltpu.VMEM((_DIM, 256), jnp.float32),
        pltpu.VMEM((_DIM, 256), jnp.float32),
        pltpu.VMEM((256, 2 * _DIM), jnp.float32),
        pltpu.VMEM((256, 2 * _DIM), jnp.float32),
        pltpu.SemaphoreType.DMA,
        pltpu.SemaphoreType.DMA,
    ],
    compiler_params=pltpu.CompilerParams(
        use_tc_tiling_on_sc=True, needs_layout_passes=False),
)
def _format(tr_hbm, ti_hbm, ct_hbm,
            br0, bi0, br1, bi1, cb0, cb1, sem_in, sem_out):
    wid = lax.axis_index("s") * _NC + lax.axis_index("c")
    iota16 = lax.iota(jnp.int32, 16)
    cvs = [iota16 + 16 * k for k in range(16)]
    br = (br0, br1)
    bi = (bi0, bi1)
    cb = (cb0, cb1)

    def win_v0(t):
        return pl.multiple_of((wid * _JPW + t * 2) * 128, 128)

    def fire_in(t, slot):
        v0 = win_v0(t)
        pltpu.async_copy(tr_hbm.at[:, pl.ds(v0, 256)], br[slot], sem_in)
        pltpu.async_copy(ti_hbm.at[:, pl.ds(v0, 256)], bi[slot], sem_in)

    def shuffle(bufr, bufi, ctb, ncols):
        nk = ncols // 16

        def dd(d, tc):
            for k in range(nk):
                sk_r = (cvs[k] + d) & 63
                sk_i = (cvs[k] + (d + _DIM)) & 63
                vr = bufr[d, pl.ds(16 * k, 16)]
                vi = bufi[d, pl.ds(16 * k, 16)]
                plsc.store_scatter(ctb, [cvs[k], sk_r], vr)
                plsc.store_scatter(ctb, [cvs[k], sk_i], vi)
            return tc

        lax.fori_loop(0, _DIM, dd, 0)

    fire_in(0, 0)

    def phase(t, slot):
        @pl.when(t + 1 < _WPW)
        def _():
            fire_in(t + 1, 1 - slot)

        pltpu.make_async_copy(
            tr_hbm.at[:, pl.ds(0, 256)], br[slot], sem_in).wait()
        pltpu.make_async_copy(
            tr_hbm.at[:, pl.ds(0, 256)], bi[slot], sem_in).wait()

        @pl.when(t >= 2)
        def _():
            pltpu.make_async_copy(
                cb[slot], ct_hbm.at[pl.ds(0, 256)], sem_out).wait()

        shuffle(br[slot], bi[slot], cb[slot], 256)
        pltpu.async_copy(
            cb[slot], ct_hbm.at[pl.ds(win_v0(t), 256)], sem_out)

    def pair(p, carry):
        phase(2 * p, 0)
        phase(2 * p + 1, 1)
        return carry

    lax.fori_loop(0, _WPW // 2, pair, 0)
    pltpu.make_async_copy(cb0, ct_hbm.at[pl.ds(0, 256)], sem_out).wait()
    pltpu.make_async_copy(cb1, ct_hbm.at[pl.ds(0, 256)], sem_out).wait()

    @pl.when(wid < _NJR + 1)
    def _():
        j = _JPW * _NW + wid
        partial = j >= _NJ
        v0 = pl.multiple_of(jnp.minimum(j, _NJ) * 128, 128)
        pltpu.sync_copy(tr_hbm.at[:, pl.ds(v0, 128)],
                        br0.at[:, pl.ds(0, 128)])
        pltpu.sync_copy(ti_hbm.at[:, pl.ds(v0, 128)],
                        bi0.at[:, pl.ds(0, 128)])
        shuffle(br0, bi0, cb0, 128)

        @pl.when(partial)
        def _():
            pltpu.sync_copy(cb0.at[pl.ds(0, 64)],
                            ct_hbm.at[pl.ds(v0, 64)])

        @pl.when(jnp.logical_not(partial))
        def _():
            pltpu.sync_copy(cb0.at[pl.ds(0, 128)],
                            ct_hbm.at[pl.ds(v0, 128)])


@functools.partial(
    pl.kernel,
    out_type=(
        jax.ShapeDtypeStruct((_HIST, _DIM // 8, _NBT, 8, 128), jnp.float32),
        jax.ShapeDtypeStruct((_HIST, _DIM // 8, _NBT, 8, 128), jnp.float32),
    ),
    mesh=_mesh,
    scratch_types=[
        pltpu.VMEM((128,), jnp.int32),
        pltpu.VMEM((128,), jnp.int32),
        pltpu.VMEM((128, 2 * _DIM), jnp.float32),
        pltpu.VMEM((128, 2 * _DIM), jnp.float32),
        pltpu.VMEM((_DIM, 128), jnp.float32),
        pltpu.VMEM((_DIM, 128), jnp.float32),
        pltpu.VMEM((_DIM, 128), jnp.float32),
        pltpu.VMEM((_DIM, 128), jnp.float32),
        pltpu.SemaphoreType.DMA,
        pltpu.SemaphoreType.DMA,
    ],
    compiler_params=pltpu.CompilerParams(
        use_tc_tiling_on_sc=False, needs_layout_passes=False),
)
def _gather(ids_hbm, ct_hbm, out_r_hbm, out_i_hbm,
            idx0, idx1, rows0, rows1, otr0, oti0, otr1, oti1,
            sem_g, sem_o):
    wid = lax.axis_index("s") * _NC + lax.axis_index("c")
    ubase = wid * _UPW
    iota16 = lax.iota(jnp.int32, 16)
    cvs = [iota16 + 16 * k for k in range(8)]
    idx = (idx0, idx1)
    rows = (rows0, rows1)
    otr = (otr0, otr1)
    oti = (oti0, oti1)

    def hb(t):
        u = ubase + t
        return u // _NBT, u % _NBT

    def fire_unit(t, slot):
        h, bt = hb(t)
        pltpu.sync_copy(ids_hbm.at[h, pl.ds(bt * 128, 128)], idx[slot])
        pltpu.async_copy(ct_hbm.at[idx[slot]], rows[slot], sem_g)

    fire_unit(0, 0)

    def phase(t, slot):
        @pl.when(t + 1 < _UPW)
        def _():
            fire_unit(t + 1, 1 - slot)

        pltpu.make_async_copy(ct_hbm.at[idx[slot]], rows[slot], sem_g).wait()

        @pl.when(t >= 2)
        def _():
            for dt in range(_DIM // 8):
                pltpu.make_async_copy(
                    otr[slot].at[pl.ds(dt * 8, 8)],
                    out_r_hbm.at[0, 0, 0], sem_o).wait()
                pltpu.make_async_copy(
                    oti[slot].at[pl.ds(dt * 8, 8)],
                    out_i_hbm.at[0, 0, 0], sem_o).wait()

        rr = rows[slot]
        tr = otr[slot]
        ti = oti[slot]
        vv = [idx[slot][pl.ds(16 * k, 16)] & 63 for k in range(8)]

        def dd(d, tc):
            for k in range(8):
                sk_r = (vv[k] + d) & 63
                sk_i = (vv[k] + (d + _DIM)) & 63
                vr = plsc.load_gather(rr, [cvs[k], sk_r])
                vi = plsc.load_gather(rr, [cvs[k], sk_i])
                tr[d, pl.ds(16 * k, 16)] = vr
                ti[d, pl.ds(16 * k, 16)] = vi
            return tc

        lax.fori_loop(0, _DIM, dd, 0)
        h, bt = hb(t)
        for dt in range(_DIM // 8):
            pltpu.async_copy(tr.at[pl.ds(dt * 8, 8)],
                             out_r_hbm.at[h, dt, bt], sem_o)
            pltpu.async_copy(ti.at[pl.ds(dt * 8, 8)],
                             out_i_hbm.at[h, dt, bt], sem_o)

    def pair(p, carry):
        phase(2 * p, 0)
        phase(2 * p + 1, 1)
        return carry

    lax.fori_loop(0, _UPW // 2, pair, 0)
    for slot in (0, 1):
        for dt in range(_DIM // 8):
            pltpu.make_async_copy(
                otr[slot].at[pl.ds(dt * 8, 8)],
                out_r_hbm.at[0, 0, 0], sem_o).wait()
            pltpu.make_async_copy(
                oti[slot].at[pl.ds(dt * 8, 8)],
                out_i_hbm.at[0, 0, 0], sem_o).wait()


def kernel(input_ids, emb_real, emb_imag):
    ids_t = input_ids.T.astype(jnp.int32)
    ct = _format(emb_real.T, emb_imag.T)
    o_r, o_i = _gather(ids_t, ct)
    p_r = o_r.transpose(0, 1, 3, 2, 4).reshape(_HIST, _DIM, _BATCH)
    p_i = o_i.transpose(0, 1, 3, 2, 4).reshape(_HIST, _DIM, _BATCH)
    out_t = lax.complex(p_r, p_i)
    return out_t.transpose(2, 0, 1)


# speedup vs baseline: 3.0936x; 1.5170x over previous
import functools

import jax
import jax.numpy as jnp
from jax import lax
from jax.experimental import pallas as pl
from jax.experimental.pallas import tpu as pltpu
from jax.experimental.pallas import tpu_sc as plsc

_VOCAB = 1000000
_DIM = 32
_BATCH = 4096
_HIST = 50

_NC = 2
_NS = 16
_NW = _NC * _NS
_NBT = _BATCH // 128
_UNITS = _HIST * _NBT
_UPW = _UNITS // _NW

_NJ = _VOCAB // 128
_JPW = _NJ // _NW
_NJR = _NJ - _JPW * _NW
_WPW = _JPW // 2
_VPAD = 1000064

_mesh = plsc.VectorSubcoreMesh(core_axis_name="c", subcore_axis_name="s")


@functools.partial(
    pl.kernel,
    out_type=jax.ShapeDtypeStruct((_VPAD * 2 * _DIM,), jnp.float32),
    mesh=_mesh,
    scratch_types=[
        pltpu.VMEM((_DIM, 256), jnp.float32),
        pltpu.VMEM((_DIM, 256), jnp.float32),
        pltpu.VMEM((_DIM, 256), jnp.float32),
        pltpu.VMEM((_DIM, 256), jnp.float32),
        pltpu.VMEM((256 * 2 * _DIM,), jnp.float32),
        pltpu.VMEM((256 * 2 * _DIM,), jnp.float32),
        pltpu.SemaphoreType.DMA,
        pltpu.SemaphoreType.DMA,
    ],
    compiler_params=pltpu.CompilerParams(
        use_tc_tiling_on_sc=True, needs_layout_passes=False),
)
def _format(tr_hbm, ti_hbm, ct_hbm,
            br0, bi0, br1, bi1, cb0, cb1, sem_in, sem_out):
    wid = lax.axis_index("s") * _NC + lax.axis_index("c")
    iota16 = lax.iota(jnp.int32, 16)
    cvs = [iota16 + 16 * k for k in range(16)]
    cv64 = [(iota16 + 16 * k) * (2 * _DIM) for k in range(16)]
    br = (br0, br1)
    bi = (bi0, bi1)
    cb = (cb0, cb1)

    def win_v0(t):
        return pl.multiple_of((wid * _JPW + t * 2) * 128, 128)

    def fire_in(t, slot):
        v0 = win_v0(t)
        pltpu.async_copy(tr_hbm.at[:, pl.ds(v0, 256)], br[slot], sem_in)
        pltpu.async_copy(ti_hbm.at[:, pl.ds(v0, 256)], bi[slot], sem_in)

    def shuffle(bufr, bufi, ctb, ncols):
        nk = ncols // 16

        def dd(d, tc):
            for k in range(nk):
                sk_r = ((cvs[k] + d) & 63) + cv64[k]
                sk_i = ((cvs[k] + (d + _DIM)) & 63) + cv64[k]
                vr = bufr[d, pl.ds(16 * k, 16)]
                vi = bufi[d, pl.ds(16 * k, 16)]
                plsc.store_scatter(ctb, [sk_r], vr)
                plsc.store_scatter(ctb, [sk_i], vi)
            return tc

        lax.fori_loop(0, _DIM, dd, 0)

    fire_in(0, 0)

    def phase(t, slot):
        @pl.when(t + 1 < _WPW)
        def _():
            fire_in(t + 1, 1 - slot)

        pltpu.make_async_copy(
            tr_hbm.at[:, pl.ds(0, 256)], br[slot], sem_in).wait()
        pltpu.make_async_copy(
            tr_hbm.at[:, pl.ds(0, 256)], bi[slot], sem_in).wait()

        @pl.when(t >= 2)
        def _():
            pltpu.make_async_copy(
                cb[slot], ct_hbm.at[pl.ds(0, 256 * 2 * _DIM)],
                sem_out).wait()

        shuffle(br[slot], bi[slot], cb[slot], 256)
        pltpu.async_copy(
            cb[slot],
            ct_hbm.at[pl.ds(win_v0(t) * 2 * _DIM, 256 * 2 * _DIM)], sem_out)

    def pair(p, carry):
        phase(2 * p, 0)
        phase(2 * p + 1, 1)
        return carry

    lax.fori_loop(0, _WPW // 2, pair, 0)
    pltpu.make_async_copy(
        cb0, ct_hbm.at[pl.ds(0, 256 * 2 * _DIM)], sem_out).wait()
    pltpu.make_async_copy(
        cb1, ct_hbm.at[pl.ds(0, 256 * 2 * _DIM)], sem_out).wait()

    @pl.when(wid < _NJR + 1)
    def _():
        j = _JPW * _NW + wid
        partial = j >= _NJ
        v0 = pl.multiple_of(jnp.minimum(j, _NJ) * 128, 128)
        pltpu.sync_copy(tr_hbm.at[:, pl.ds(v0, 128)],
                        br0.at[:, pl.ds(0, 128)])
        pltpu.sync_copy(ti_hbm.at[:, pl.ds(v0, 128)],
                        bi0.at[:, pl.ds(0, 128)])
        shuffle(br0, bi0, cb0, 128)

        half = 64 * 2 * _DIM

        @pl.when(partial)
        def _():
            pltpu.sync_copy(cb0.at[pl.ds(0, half)],
                            ct_hbm.at[pl.ds(v0 * 2 * _DIM, half)])

        @pl.when(jnp.logical_not(partial))
        def _():
            pltpu.sync_copy(cb0.at[pl.ds(0, 2 * half)],
                            ct_hbm.at[pl.ds(v0 * 2 * _DIM, 2 * half)])


@functools.partial(
    pl.kernel,
    out_type=(
        jax.ShapeDtypeStruct((_HIST, _DIM // 8, _NBT, 8, 128), jnp.float32),
        jax.ShapeDtypeStruct((_HIST, _DIM // 8, _NBT, 8, 128), jnp.float32),
    ),
    mesh=_mesh,
    scratch_types=[
        pltpu.VMEM((128,), jnp.int32),
        pltpu.VMEM((128,), jnp.int32),
        pltpu.VMEM((128, 2 * _DIM), jnp.float32),
        pltpu.VMEM((128, 2 * _DIM), jnp.float32),
        pltpu.VMEM((_DIM, 128), jnp.float32),
        pltpu.VMEM((_DIM, 128), jnp.float32),
        pltpu.VMEM((_DIM, 128), jnp.float32),
        pltpu.VMEM((_DIM, 128), jnp.float32),
        pltpu.SemaphoreType.DMA,
        pltpu.SemaphoreType.DMA,
    ],
    compiler_params=pltpu.CompilerParams(
        use_tc_tiling_on_sc=False, needs_layout_passes=False),
)
def _gather(ids_hbm, ct_hbm, out_r_hbm, out_i_hbm,
            idx0, idx1, rows0, rows1, otr0, oti0, otr1, oti1,
            sem_g, sem_o):
    wid = lax.axis_index("s") * _NC + lax.axis_index("c")
    ubase = wid * _UPW
    iota16 = lax.iota(jnp.int32, 16)
    cvs = [iota16 + 16 * k for k in range(8)]
    idx = (idx0, idx1)
    rows = (rows0, rows1)
    otr = (otr0, otr1)
    oti = (oti0, oti1)

    def hb(t):
        u = ubase + t
        return u // _NBT, u % _NBT

    def fire_unit(t, slot):
        h, bt = hb(t)
        pltpu.sync_copy(ids_hbm.at[h, pl.ds(bt * 128, 128)], idx[slot])
        pltpu.async_copy(ct_hbm.at[idx[slot]], rows[slot], sem_g)

    fire_unit(0, 0)

    def phase(t, slot):
        @pl.when(t + 1 < _UPW)
        def _():
            fire_unit(t + 1, 1 - slot)

        pltpu.make_async_copy(ct_hbm.at[idx[slot]], rows[slot], sem_g).wait()

        @pl.when(t >= 2)
        def _():
            for dt in range(_DIM // 8):
                pltpu.make_async_copy(
                    otr[slot].at[pl.ds(dt * 8, 8)],
                    out_r_hbm.at[0, 0, 0], sem_o).wait()
                pltpu.make_async_copy(
                    oti[slot].at[pl.ds(dt * 8, 8)],
                    out_i_hbm.at[0, 0, 0], sem_o).wait()

        rr = rows[slot]
        tr = otr[slot]
        ti = oti[slot]
        vv = [idx[slot][pl.ds(16 * k, 16)] & 63 for k in range(8)]

        def dd(d, tc):
            for k in range(8):
                sk_r = (vv[k] + d) & 63
                sk_i = (vv[k] + (d + _DIM)) & 63
                vr = plsc.load_gather(rr, [cvs[k], sk_r])
                vi = plsc.load_gather(rr, [cvs[k], sk_i])
                tr[d, pl.ds(16 * k, 16)] = vr
                ti[d, pl.ds(16 * k, 16)] = vi
            return tc

        lax.fori_loop(0, _DIM, dd, 0)
        h, bt = hb(t)
        for dt in range(_DIM // 8):
            pltpu.async_copy(tr.at[pl.ds(dt * 8, 8)],
                             out_r_hbm.at[h, dt, bt], sem_o)
            pltpu.async_copy(ti.at[pl.ds(dt * 8, 8)],
                             out_i_hbm.at[h, dt, bt], sem_o)

    def pair(p, carry):
        phase(2 * p, 0)
        phase(2 * p + 1, 1)
        return carry

    lax.fori_loop(0, _UPW // 2, pair, 0)
    for slot in (0, 1):
        for dt in range(_DIM // 8):
            pltpu.make_async_copy(
                otr[slot].at[pl.ds(dt * 8, 8)],
                out_r_hbm.at[0, 0, 0], sem_o).wait()
            pltpu.make_async_copy(
                oti[slot].at[pl.ds(dt * 8, 8)],
                out_i_hbm.at[0, 0, 0], sem_o).wait()


def kernel(input_ids, emb_real, emb_imag):
    ids_t = input_ids.T.astype(jnp.int32)
    ct_flat = _format(emb_real.T, emb_imag.T)
    ct = ct_flat.reshape(_VPAD, 2 * _DIM)
    o_r, o_i = _gather(ids_t, ct)
    p_r = o_r.transpose(0, 1, 3, 2, 4).reshape(_HIST, _DIM, _BATCH)
    p_i = o_i.transpose(0, 1, 3, 2, 4).reshape(_HIST, _DIM, _BATCH)
    out_t = lax.complex(p_r, p_i)
    return out_t.transpose(2, 0, 1)


# speedup vs baseline: 3.0977x; 1.0013x over previous
import functools

import jax
import jax.numpy as jnp
from jax import lax
from jax.experimental import pallas as pl
from jax.experimental.pallas import tpu as pltpu
from jax.experimental.pallas import tpu_sc as plsc

_VOCAB = 1000000
_DIM = 32
_BATCH = 4096
_HIST = 50

_NC = 2
_NS = 16
_NW = _NC * _NS
_NBT = _BATCH // 128
_UNITS = _HIST * _NBT
_UPW = _UNITS // _NW

_NJ = _VOCAB // 128
_JPW = _NJ // _NW
_NJR = _NJ - _JPW * _NW
_WPW = _JPW // 2
_VPAD = 1000064

_mesh = plsc.VectorSubcoreMesh(core_axis_name="c", subcore_axis_name="s")


@functools.partial(
    pl.kernel,
    out_type=jax.ShapeDtypeStruct((_VPAD * 2 * _DIM,), jnp.float32),
    mesh=_mesh,
    scratch_types=[
        pltpu.VMEM((_DIM, 256), jnp.float32),
        pltpu.VMEM((_DIM, 256), jnp.float32),
        pltpu.VMEM((_DIM, 256), jnp.float32),
        pltpu.VMEM((_DIM, 256), jnp.float32),
        pltpu.VMEM((256 * 2 * _DIM,), jnp.float32),
        pltpu.VMEM((256 * 2 * _DIM,), jnp.float32),
        pltpu.SemaphoreType.DMA,
        pltpu.SemaphoreType.DMA,
    ],
    compiler_params=pltpu.CompilerParams(
        use_tc_tiling_on_sc=True, needs_layout_passes=False),
)
def _format(tr_hbm, ti_hbm, ct_hbm,
            br0, bi0, br1, bi1, cb0, cb1, sem_in, sem_out):
    wid = lax.axis_index("s") * _NC + lax.axis_index("c")
    iota16 = lax.iota(jnp.int32, 16)
    cvs = [iota16 + 16 * k for k in range(16)]
    cv64 = [(iota16 + 16 * k) * (2 * _DIM) for k in range(16)]
    br = (br0, br1)
    bi = (bi0, bi1)
    cb = (cb0, cb1)

    def win_v0(t):
        return pl.multiple_of((wid * _JPW + t * 2) * 128, 128)

    def fire_in(t, slot):
        v0 = win_v0(t)
        pltpu.async_copy(tr_hbm.at[:, pl.ds(v0, 256)], br[slot], sem_in)
        pltpu.async_copy(ti_hbm.at[:, pl.ds(v0, 256)], bi[slot], sem_in)

    def shuffle(bufr, bufi, ctb, ncols):
        nk = ncols // 16

        def dd(d, tc):
            for k in range(nk):
                sk_r = ((cvs[k] + d) & 63) + cv64[k]
                sk_i = sk_r ^ _DIM
                vr = bufr[d, pl.ds(16 * k, 16)]
                vi = bufi[d, pl.ds(16 * k, 16)]
                plsc.store_scatter(ctb, [sk_r], vr)
                plsc.store_scatter(ctb, [sk_i], vi)
            return tc

        lax.fori_loop(0, _DIM, dd, 0)

    fire_in(0, 0)

    def phase(t, slot):
        @pl.when(t + 1 < _WPW)
        def _():
            fire_in(t + 1, 1 - slot)

        pltpu.make_async_copy(
            tr_hbm.at[:, pl.ds(0, 256)], br[slot], sem_in).wait()
        pltpu.make_async_copy(
            tr_hbm.at[:, pl.ds(0, 256)], bi[slot], sem_in).wait()

        @pl.when(t >= 2)
        def _():
            pltpu.make_async_copy(
                cb[slot], ct_hbm.at[pl.ds(0, 256 * 2 * _DIM)],
                sem_out).wait()

        shuffle(br[slot], bi[slot], cb[slot], 256)
        pltpu.async_copy(
            cb[slot],
            ct_hbm.at[pl.ds(win_v0(t) * 2 * _DIM, 256 * 2 * _DIM)], sem_out)

    def pair(p, carry):
        phase(2 * p, 0)
        phase(2 * p + 1, 1)
        return carry

    lax.fori_loop(0, _WPW // 2, pair, 0)
    pltpu.make_async_copy(
        cb0, ct_hbm.at[pl.ds(0, 256 * 2 * _DIM)], sem_out).wait()
    pltpu.make_async_copy(
        cb1, ct_hbm.at[pl.ds(0, 256 * 2 * _DIM)], sem_out).wait()

    @pl.when(wid < _NJR + 1)
    def _():
        j = _JPW * _NW + wid
        partial = j >= _NJ
        v0 = pl.multiple_of(jnp.minimum(j, _NJ) * 128, 128)
        pltpu.sync_copy(tr_hbm.at[:, pl.ds(v0, 128)],
                        br0.at[:, pl.ds(0, 128)])
        pltpu.sync_copy(ti_hbm.at[:, pl.ds(v0, 128)],
                        bi0.at[:, pl.ds(0, 128)])
        shuffle(br0, bi0, cb0, 128)

        half = 64 * 2 * _DIM

        @pl.when(partial)
        def _():
            pltpu.sync_copy(cb0.at[pl.ds(0, half)],
                            ct_hbm.at[pl.ds(v0 * 2 * _DIM, half)])

        @pl.when(jnp.logical_not(partial))
        def _():
            pltpu.sync_copy(cb0.at[pl.ds(0, 2 * half)],
                            ct_hbm.at[pl.ds(v0 * 2 * _DIM, 2 * half)])


@functools.partial(
    pl.kernel,
    out_type=(
        jax.ShapeDtypeStruct((_HIST, _DIM // 8, _NBT, 8, 128), jnp.float32),
        jax.ShapeDtypeStruct((_HIST, _DIM // 8, _NBT, 8, 128), jnp.float32),
    ),
    mesh=_mesh,
    scratch_types=[
        pltpu.VMEM((128,), jnp.int32),
        pltpu.VMEM((128,), jnp.int32),
        pltpu.VMEM((128, 2 * _DIM), jnp.float32),
        pltpu.VMEM((128, 2 * _DIM), jnp.float32),
        pltpu.VMEM((_DIM, 128), jnp.float32),
        pltpu.VMEM((_DIM, 128), jnp.float32),
        pltpu.VMEM((_DIM, 128), jnp.float32),
        pltpu.VMEM((_DIM, 128), jnp.float32),
        pltpu.SemaphoreType.DMA,
        pltpu.SemaphoreType.DMA,
    ],
    compiler_params=pltpu.CompilerParams(
        use_tc_tiling_on_sc=False, needs_layout_passes=False),
)
def _gather(ids_hbm, ct_hbm, out_r_hbm, out_i_hbm,
            idx0, idx1, rows0, rows1, otr0, oti0, otr1, oti1,
            sem_g, sem_o):
    wid = lax.axis_index("s") * _NC + lax.axis_index("c")
    ubase = wid * _UPW
    iota16 = lax.iota(jnp.int32, 16)
    cvs = [iota16 + 16 * k for k in range(8)]
    idx = (idx0, idx1)
    rows = (rows0, rows1)
    otr = (otr0, otr1)
    oti = (oti0, oti1)

    def hb(t):
        u = ubase + t
        return u // _NBT, u % _NBT

    def fire_unit(t, slot):
        h, bt = hb(t)
        pltpu.sync_copy(ids_hbm.at[h, pl.ds(bt * 128, 128)], idx[slot])
        pltpu.async_copy(ct_hbm.at[idx[slot]], rows[slot], sem_g)

    fire_unit(0, 0)

    def phase(t, slot):
        @pl.when(t + 1 < _UPW)
        def _():
            fire_unit(t + 1, 1 - slot)

        pltpu.make_async_copy(ct_hbm.at[idx[slot]], rows[slot], sem_g).wait()

        @pl.when(t >= 2)
        def _():
            for dt in range(_DIM // 8):
                pltpu.make_async_copy(
                    otr[slot].at[pl.ds(dt * 8, 8)],
                    out_r_hbm.at[0, 0, 0], sem_o).wait()
                pltpu.make_async_copy(
                    oti[slot].at[pl.ds(dt * 8, 8)],
                    out_i_hbm.at[0, 0, 0], sem_o).wait()

        rr = rows[slot]
        tr = otr[slot]
        ti = oti[slot]
        vv = [idx[slot][pl.ds(16 * k, 16)] & 63 for k in range(8)]

        def dd(d, tc):
            for k in range(8):
                sk_r = (vv[k] + d) & 63
                sk_i = sk_r ^ _DIM
                vr = plsc.load_gather(rr, [cvs[k], sk_r])
                vi = plsc.load_gather(rr, [cvs[k], sk_i])
                tr[d, pl.ds(16 * k, 16)] = vr
                ti[d, pl.ds(16 * k, 16)] = vi
            return tc

        lax.fori_loop(0, _DIM, dd, 0)
        h, bt = hb(t)
        for dt in range(_DIM // 8):
            pltpu.async_copy(tr.at[pl.ds(dt * 8, 8)],
                             out_r_hbm.at[h, dt, bt], sem_o)
            pltpu.async_copy(ti.at[pl.ds(dt * 8, 8)],
                             out_i_hbm.at[h, dt, bt], sem_o)

    def pair(p, carry):
        phase(2 * p, 0)
        phase(2 * p + 1, 1)
        return carry

    lax.fori_loop(0, _UPW // 2, pair, 0)
    for slot in (0, 1):
        for dt in range(_DIM // 8):
            pltpu.make_async_copy(
                otr[slot].at[pl.ds(dt * 8, 8)],
                out_r_hbm.at[0, 0, 0], sem_o).wait()
            pltpu.make_async_copy(
                oti[slot].at[pl.ds(dt * 8, 8)],
                out_i_hbm.at[0, 0, 0], sem_o).wait()


def kernel(input_ids, emb_real, emb_imag):
    ids_t = input_ids.T.astype(jnp.int32)
    ct_flat = _format(emb_real.T, emb_imag.T)
    ct = ct_flat.reshape(_VPAD, 2 * _DIM)
    o_r, o_i = _gather(ids_t, ct)
    p_r = o_r.transpose(0, 1, 3, 2, 4).reshape(_HIST, _DIM, _BATCH)
    p_i = o_i.transpose(0, 1, 3, 2, 4).reshape(_HIST, _DIM, _BATCH)
    out_t = lax.complex(p_r, p_i)
    return out_t.transpose(2, 0, 1)
